# baseline (speedup 1.0000x reference)
import functools

import jax
import jax.numpy as jnp
from jax import lax
from jax.experimental import pallas as pl
from jax.experimental.pallas import tpu as pltpu
from jax.experimental.pallas import tpu_sc as plsc

N = 10000
E = 320000
IN = 128
OUT = 128
R = 65

NC = 2
NS = 16
NW = NC * NS
SUB = 80
EPW = E // NW
STG = 2000
N_ST = EPW // STG
ROWS_N = 624
TAIL = N - NS * ROWS_N
ZR = 16


def _h_body(x_ref, coef_ref, basis_ref, g_ref, rel_ref,
            h_ref, flat_ref, dst_ref):
    r = pl.program_id(0)

    @pl.when(r == 0)
    def _():
        flat_ref[...] = rel_ref[...] * N + g_ref[0]
        dst_ref[...] = g_ref[1]

    w = jnp.dot(coef_ref[pl.ds(r, 1), :], basis_ref[...],
                preferred_element_type=jnp.float32)
    h_ref[0] = jnp.dot(x_ref[...], w.reshape(IN, OUT),
                       preferred_element_type=jnp.float32)


def _h_all(x, coef, basis2, g, rel):
    return pl.pallas_call(
        _h_body,
        grid=(R,),
        in_specs=[
            pl.BlockSpec((N, IN), lambda r: (0, 0)),
            pl.BlockSpec((R, R), lambda r: (0, 0)),
            pl.BlockSpec((R, IN * OUT), lambda r: (0, 0)),
            pl.BlockSpec((2, E), lambda r: (0, 0)),
            pl.BlockSpec((E,), lambda r: (0,)),
        ],
        out_specs=[
            pl.BlockSpec((1, N, OUT), lambda r: (r, 0, 0)),
            pl.BlockSpec((E,), lambda r: (0,)),
            pl.BlockSpec((E,), lambda r: (0,)),
        ],
        out_shape=(
            jax.ShapeDtypeStruct((R, N, OUT), jnp.float32),
            jax.ShapeDtypeStruct((E,), jnp.int32),
            jax.ShapeDtypeStruct((E,), jnp.int32),
        ),
    )(x, coef, basis2, g, rel)


def _sc_edge_body(h_hbm, flat_hbm, dst_hbm, part_hbm,
                  dst_v, flat_v, rows_a, rows_b, rows_c, rows_d,
                  acc, zblk, gsem_a, gsem_b, gsem_c, gsem_d,
                  ssem_a, ssem_b, ssem_c, ssem_d):
    cid = lax.axis_index("c")
    sid = lax.axis_index("s")
    wid = cid * NS + sid

    z = jnp.zeros((1, 16), jnp.float32)

    @pl.loop(0, ZR)
    def _(i):
        @pl.loop(0, OUT, step=16)
        def _(j):
            zblk[pl.ds(i, 1), pl.ds(j, 16)] = z

    @pl.loop(0, ROWS_N // ZR)
    def _(k):
        pltpu.sync_copy(zblk, acc.at[pl.ds(sid * ROWS_N + k * ZR, ZR)])

    @pl.when(sid == NS - 1)
    def _():
        pltpu.sync_copy(zblk.at[pl.ds(0, TAIL)],
                        acc.at[pl.ds(NS * ROWS_N, TAIL)])

    plsc.subcore_barrier()

    @pl.loop(0, N_ST)
    def _(st):
        base = wid * EPW + st * STG
        pltpu.sync_copy(flat_hbm.at[pl.ds(base, STG)], flat_v)
        pltpu.sync_copy(dst_hbm.at[pl.ds(base, STG)], dst_v)

        rows = (rows_a, rows_b, rows_c, rows_d)
        gsem = (gsem_a, gsem_b, gsem_c, gsem_d)
        ssem = (ssem_a, ssem_b, ssem_c, ssem_d)
        NSUB = STG // SUB

        def g_slice(j):
            return h_hbm.at[flat_v.at[pl.ds(j * SUB, SUB)]]

        def s_slice(j):
            return acc.at[dst_v.at[pl.ds(j * SUB, SUB)]]

        pltpu.async_copy(g_slice(0), rows[0], gsem[0])
        pltpu.async_copy(g_slice(1), rows[1], gsem[1])
        pltpu.async_copy(g_slice(2), rows[2], gsem[2])

        @pl.loop(0, (NSUB - 1) // 4)
        def _(k):
            for t in range(4):
                j = 4 * k + t
                nb = (t + 3) % 4
                pltpu.make_async_copy(g_slice(j), rows[t], gsem[t]).wait()
                pltpu.async_copy(rows[t], s_slice(j), ssem[t], add=True)
                if t == 0:
                    @pl.when(k > 0)
                    def _():
                        pltpu.make_async_copy(rows[nb], s_slice(j),
                                              ssem[nb]).wait()
                else:
                    pltpu.make_async_copy(rows[nb], s_slice(j),
                                          ssem[nb]).wait()

                @pl.when(j + 3 < NSUB)
                def _():
                    pltpu.async_copy(g_slice(j + 3), rows[nb], gsem[nb])

        jt = NSUB - 1
        pltpu.make_async_copy(g_slice(jt), rows[jt % 4], gsem[jt % 4]).wait()
        pltpu.async_copy(rows[jt % 4], s_slice(jt), ssem[jt % 4], add=True)
        pltpu.make_async_copy(rows[3], s_slice(jt), ssem[3]).wait()
        pltpu.make_async_copy(rows[0], s_slice(jt), ssem[0]).wait()

    plsc.subcore_barrier()

    off = sid * ROWS_N
    pltpu.sync_copy(acc.at[pl.ds(off, ROWS_N)],
                    part_hbm.at[cid, pl.ds(off, ROWS_N)])

    @pl.when(sid == NS - 1)
    def _():
        pltpu.sync_copy(acc.at[pl.ds(NS * ROWS_N, TAIL)],
                        part_hbm.at[cid, pl.ds(NS * ROWS_N, TAIL)])


def _sc_edges(h2, flat, dst):
    mesh = plsc.VectorSubcoreMesh(core_axis_name="c", subcore_axis_name="s")
    kern = pl.kernel(
        _sc_edge_body,
        out_type=jax.ShapeDtypeStruct((NC, N, OUT), jnp.float32),
        mesh=mesh,
        scratch_types=[
            pltpu.VMEM((STG,), jnp.int32),
            pltpu.VMEM((STG,), jnp.int32),
            pltpu.VMEM((SUB, OUT), jnp.float32),
            pltpu.VMEM((SUB, OUT), jnp.float32),
            pltpu.VMEM((SUB, OUT), jnp.float32),
            pltpu.VMEM((SUB, OUT), jnp.float32),
            pltpu.VMEM_SHARED((N, OUT), jnp.float32),
            pltpu.VMEM((ZR, OUT), jnp.float32),
            pltpu.SemaphoreType.DMA,
            pltpu.SemaphoreType.DMA,
            pltpu.SemaphoreType.DMA,
            pltpu.SemaphoreType.DMA,
            pltpu.SemaphoreType.DMA,
            pltpu.SemaphoreType.DMA,
            pltpu.SemaphoreType.DMA,
            pltpu.SemaphoreType.DMA,
        ],
    )
    return kern(h2, flat, dst)


def _fin_body(p_ref, b_ref, o_ref):
    o_ref[...] = jnp.maximum(p_ref[0] + p_ref[1] + b_ref[...], 0.0)


def _finalize(part, bias2):
    return pl.pallas_call(
        _fin_body,
        out_shape=jax.ShapeDtypeStruct((N, OUT), jnp.float32),
    )(part, bias2)


def kernel(g, node_feats, edge_feats, basis, coef, bias):
    basis2 = basis.reshape(R, IN * OUT)
    h, flat, dst = _h_all(node_feats, coef, basis2, g, edge_feats)
    h2 = h.reshape(R * N, OUT)
    part = _sc_edges(h2, flat, dst)
    bias2 = bias.reshape(1, OUT)
    return _finalize(part, bias2)


# speedup vs baseline: 1.0352x; 1.0352x over previous
import functools

import jax
import jax.numpy as jnp
from jax import lax
from jax.experimental import pallas as pl
from jax.experimental.pallas import tpu as pltpu
from jax.experimental.pallas import tpu_sc as plsc

N = 10000
E = 320000
IN = 128
OUT = 128
R = 65

NC = 2
NS = 16
NW = NC * NS
SUB = 80
EPW = E // NW
STG = 2000
N_ST = EPW // STG
ROWS_N = 624
TAIL = N - NS * ROWS_N
ZR = 16


def _prep_body(coef_ref, basis_ref, g_ref, rel_ref,
               w_ref, flat_ref, dst_ref):
    w_ref[...] = jnp.dot(coef_ref[...], basis_ref[...],
                         preferred_element_type=jnp.float32)
    flat_ref[...] = rel_ref[...] * N + g_ref[0]
    dst_ref[...] = g_ref[1]


def _prep(coef, basis2, g, rel):
    return pl.pallas_call(
        _prep_body,
        out_shape=(
            jax.ShapeDtypeStruct((R, IN * OUT), jnp.float32),
            jax.ShapeDtypeStruct((E,), jnp.int32),
            jax.ShapeDtypeStruct((E,), jnp.int32),
        ),
    )(coef, basis2, g, rel)


def _h_body(x_ref, w_ref, h_ref):
    r = pl.program_id(0)
    h_ref[0] = jnp.dot(x_ref[...], w_ref[pl.ds(r, 1), :].reshape(IN, OUT),
                       preferred_element_type=jnp.float32)


def _h_all(x, w2):
    return pl.pallas_call(
        _h_body,
        grid=(R,),
        in_specs=[
            pl.BlockSpec((N, IN), lambda r: (0, 0)),
            pl.BlockSpec((R, IN * OUT), lambda r: (0, 0)),
        ],
        out_specs=pl.BlockSpec((1, N, OUT), lambda r: (r, 0, 0)),
        out_shape=jax.ShapeDtypeStruct((R, N, OUT), jnp.float32),
    )(x, w2)


def _sc_edge_body(h_hbm, flat_hbm, dst_hbm, part_hbm,
                  dst_v, flat_v, rows_a, rows_b, rows_c, rows_d,
                  acc, zblk, gsem_a, gsem_b, gsem_c, gsem_d,
                  ssem_a, ssem_b, ssem_c, ssem_d):
    cid = lax.axis_index("c")
    sid = lax.axis_index("s")
    wid = cid * NS + sid

    z = jnp.zeros((1, 16), jnp.float32)

    @pl.loop(0, ZR)
    def _(i):
        @pl.loop(0, OUT, step=16)
        def _(j):
            zblk[pl.ds(i, 1), pl.ds(j, 16)] = z

    @pl.loop(0, ROWS_N // ZR)
    def _(k):
        pltpu.sync_copy(zblk, acc.at[pl.ds(sid * ROWS_N + k * ZR, ZR)])

    @pl.when(sid == NS - 1)
    def _():
        pltpu.sync_copy(zblk.at[pl.ds(0, TAIL)],
                        acc.at[pl.ds(NS * ROWS_N, TAIL)])

    plsc.subcore_barrier()

    @pl.loop(0, N_ST)
    def _(st):
        base = wid * EPW + st * STG
        pltpu.sync_copy(flat_hbm.at[pl.ds(base, STG)], flat_v)
        pltpu.sync_copy(dst_hbm.at[pl.ds(base, STG)], dst_v)

        rows = (rows_a, rows_b, rows_c, rows_d)
        gsem = (gsem_a, gsem_b, gsem_c, gsem_d)
        ssem = (ssem_a, ssem_b, ssem_c, ssem_d)
        NSUB = STG // SUB

        def g_slice(j):
            return h_hbm.at[flat_v.at[pl.ds(j * SUB, SUB)]]

        def s_slice(j):
            return acc.at[dst_v.at[pl.ds(j * SUB, SUB)]]

        pltpu.async_copy(g_slice(0), rows[0], gsem[0])
        pltpu.async_copy(g_slice(1), rows[1], gsem[1])
        pltpu.async_copy(g_slice(2), rows[2], gsem[2])

        @pl.loop(0, (NSUB - 1) // 4)
        def _(k):
            for t in range(4):
                j = 4 * k + t
                nb = (t + 3) % 4
                pltpu.make_async_copy(g_slice(j), rows[t], gsem[t]).wait()
                pltpu.async_copy(rows[t], s_slice(j), ssem[t], add=True)
                if t == 0:
                    @pl.when(k > 0)
                    def _():
                        pltpu.make_async_copy(rows[nb], s_slice(j),
                                              ssem[nb]).wait()
                else:
                    pltpu.make_async_copy(rows[nb], s_slice(j),
                                          ssem[nb]).wait()

                @pl.when(j + 3 < NSUB)
                def _():
                    pltpu.async_copy(g_slice(j + 3), rows[nb], gsem[nb])

        jt = NSUB - 1
        pltpu.make_async_copy(g_slice(jt), rows[jt % 4], gsem[jt % 4]).wait()
        pltpu.async_copy(rows[jt % 4], s_slice(jt), ssem[jt % 4], add=True)
        pltpu.make_async_copy(rows[3], s_slice(jt), ssem[3]).wait()
        pltpu.make_async_copy(rows[0], s_slice(jt), ssem[0]).wait()

    plsc.subcore_barrier()

    off = sid * ROWS_N
    pltpu.sync_copy(acc.at[pl.ds(off, ROWS_N)],
                    part_hbm.at[cid, pl.ds(off, ROWS_N)])

    @pl.when(sid == NS - 1)
    def _():
        pltpu.sync_copy(acc.at[pl.ds(NS * ROWS_N, TAIL)],
                        part_hbm.at[cid, pl.ds(NS * ROWS_N, TAIL)])


def _sc_edges(h2, flat, dst):
    mesh = plsc.VectorSubcoreMesh(core_axis_name="c", subcore_axis_name="s")
    kern = pl.kernel(
        _sc_edge_body,
        out_type=jax.ShapeDtypeStruct((NC, N, OUT), jnp.float32),
        mesh=mesh,
        scratch_types=[
            pltpu.VMEM((STG,), jnp.int32),
            pltpu.VMEM((STG,), jnp.int32),
            pltpu.VMEM((SUB, OUT), jnp.float32),
            pltpu.VMEM((SUB, OUT), jnp.float32),
            pltpu.VMEM((SUB, OUT), jnp.float32),
            pltpu.VMEM((SUB, OUT), jnp.float32),
            pltpu.VMEM_SHARED((N, OUT), jnp.float32),
            pltpu.VMEM((ZR, OUT), jnp.float32),
            pltpu.SemaphoreType.DMA,
            pltpu.SemaphoreType.DMA,
            pltpu.SemaphoreType.DMA,
            pltpu.SemaphoreType.DMA,
            pltpu.SemaphoreType.DMA,
            pltpu.SemaphoreType.DMA,
            pltpu.SemaphoreType.DMA,
            pltpu.SemaphoreType.DMA,
        ],
    )
    return kern(h2, flat, dst)


def _fin_body(p_ref, b_ref, o_ref):
    o_ref[...] = jnp.maximum(p_ref[0] + p_ref[1] + b_ref[...], 0.0)


def _finalize(part, bias2):
    return pl.pallas_call(
        _fin_body,
        out_shape=jax.ShapeDtypeStruct((N, OUT), jnp.float32),
    )(part, bias2)


def kernel(g, node_feats, edge_feats, basis, coef, bias):
    basis2 = basis.reshape(R, IN * OUT)
    w2, flat, dst = _prep(coef, basis2, g, edge_feats)
    h = _h_all(node_feats, w2)
    h2 = h.reshape(R * N, OUT)
    part = _sc_edges(h2, flat, dst)
    bias2 = bias.reshape(1, OUT)
    return _finalize(part, bias2)
